# baseline (speedup 1.0000x reference)
import jax
import jax.numpy as jnp
from jax import lax
from jax.experimental import pallas as pl
from jax.experimental.pallas import tpu as pltpu
from jax.experimental.pallas import tpu_sc as plsc

_N_NODES = 10000
_N_GATHER = 320000
_N_SEG = 8192
_D = 128

_NC = 2
_NS = 16
_NW = _NC * _NS
_SEG_PER_W = _N_SEG // _NW
_CHUNK = 128
_L = 16
_G = _CHUNK // _L

_INT_MAX = 2**31 - 1


def _iota16():
    return lax.iota(jnp.int32, _L)


def _lower_bound(seg_hbm, probe_ref, pidx_ref, sem, t_splat):
    lo = 0
    for s16 in (65536, 4096, 256, 16, 1):
        probe = lo + (_iota16() + 1) * s16 - 1
        pidx_ref[...] = jnp.minimum(probe, _N_GATHER - 1)
        pltpu.async_copy(seg_hbm.at[pidx_ref], probe_ref, sem).wait()
        vals = jnp.where(probe < _N_GATHER, probe_ref[...],
                         jnp.full((_L,), _INT_MAX, jnp.int32))
        c = jnp.sum((vals < t_splat).astype(jnp.int32))
        lo = lo + c * s16
    return lo


def _segmax_body(x_hbm, gid_hbm, seg_hbm, out_hbm,
                 gid0, gid1, seg0, seg1, rows0, rows1, segst, acc,
                 probe, pidx, ssem, gsem0, gsem1, isem0, isem1):
    cid = lax.axis_index("c")
    sid = lax.axis_index("s")
    wid = sid * _NC + cid
    base = wid * _SEG_PER_W
    base_v = jnp.full((_L,), base, jnp.int32)

    gid_b = (gid0, gid1)
    seg_b = (seg0, seg1)
    rows_b = (rows0, rows1)
    gsem = (gsem0, gsem1)
    isem = (isem0, isem1)

    neg = jnp.full((_L,), -jnp.inf, jnp.float32)

    def init_body(i, _):
        for j in range(_D // _L):
            acc[i, pl.ds(j * _L, _L)] = neg
        return 0

    lax.fori_loop(0, _SEG_PER_W, init_body, 0)

    lo = _lower_bound(seg_hbm, probe, pidx, ssem, base_v)
    hi = _lower_bound(seg_hbm, probe, pidx, ssem, base_v + _SEG_PER_W)
    lo_al = pl.multiple_of((lo // 8) * 8, 8)
    nch = (hi - lo_al + (_CHUNK - 1)) // _CHUNK

    def r0_of(c):
        return pl.multiple_of(
            jnp.minimum(lo_al + c * _CHUNK, _N_GATHER - _CHUNK), 8)

    def issue_idx(c, b):
        r0 = r0_of(c)
        pltpu.async_copy(gid_hbm.at[pl.ds(r0, _CHUNK)], gid_b[b], isem[b])
        pltpu.async_copy(seg_hbm.at[pl.ds(r0, _CHUNK)], seg_b[b], isem[b])

    def wait_idx(c, b):
        r0 = r0_of(c)
        pltpu.make_async_copy(gid_hbm.at[pl.ds(r0, _CHUNK)], gid_b[b],
                              isem[b]).wait()
        pltpu.make_async_copy(seg_hbm.at[pl.ds(r0, _CHUNK)], seg_b[b],
                              isem[b]).wait()

    def issue_gather(b):
        pltpu.async_copy(x_hbm.at[gid_b[b]], rows_b[b], gsem[b])

    def wait_gather(b):
        pltpu.make_async_copy(x_hbm.at[gid_b[b]], rows_b[b], gsem[b]).wait()

    @pl.when(nch > 0)
    def _prime0():
        issue_idx(0, 0)

    @pl.when(nch > 1)
    def _prime1():
        issue_idx(1, 1)

    @pl.when(nch > 0)
    def _prime2():
        wait_idx(0, 0)
        issue_gather(0)

    def compute(rows, safe):
        def group_body(g, _):
            gb = g * _L
            seg_vec = segst[pl.ds(gb, _L)]
            s0 = seg_vec[0]
            s15 = seg_vec[_L - 1]
            off0 = s0 - base

            if safe:
                fast_cond = s0 == s15
            else:
                fast_cond = ((s0 == s15) & (off0 >= 0)
                             & (off0 < _SEG_PER_W))

            @pl.when(fast_cond)
            def _fast():
                for j in range(_D // _L):
                    js = pl.ds(j * _L, _L)
                    vs = [rows[gb + r, js] for r in range(_L)]
                    while len(vs) > 1:
                        vs = [jnp.maximum(vs[i], vs[i + 1])
                              for i in range(0, len(vs), 2)]
                    acc[off0, js] = jnp.maximum(acc[off0, js], vs[0])

            @pl.when(s0 != s15)
            def _slow():
                for r in range(_L):
                    off = seg_vec[r] - base

                    def _row(r=r, off=off):
                        for j in range(_D // _L):
                            js = pl.ds(j * _L, _L)
                            acc[off, js] = jnp.maximum(acc[off, js],
                                                       rows[gb + r, js])

                    if safe:
                        _row()
                    else:
                        pl.when((off >= 0) & (off < _SEG_PER_W))(_row)

            return 0

        lax.fori_loop(0, _G, group_body, 0)

    def pair_body(cc, _):
        for b in (0, 1):
            c = cc * 2 + b

            @pl.when(c < nch)
            def _iter(c=c, b=b):
                nb = 1 - b
                wait_gather(b)

                @pl.when(c + 1 < nch)
                def _wi():
                    wait_idx(c + 1, nb)

                for g in range(_G):
                    segst[pl.ds(g * _L, _L)] = seg_b[b][pl.ds(g * _L, _L)]

                @pl.when(c + 1 < nch)
                def _ig():
                    issue_gather(nb)

                @pl.when(c + 2 < nch)
                def _ii():
                    issue_idx(c + 2, b)

                safe_c = (c > 0) & (lo_al + (c + 1) * _CHUNK <= hi)

                @pl.when(safe_c)
                def _cs():
                    compute(rows_b[b], True)

                @pl.when(jnp.logical_not(safe_c))
                def _ce():
                    compute(rows_b[b], False)

        return 0

    lax.fori_loop(0, (nch + 1) // 2, pair_body, 0)

    pltpu.sync_copy(acc, out_hbm.at[pl.ds(base, _SEG_PER_W)])


@jax.jit
def _segmax(x, gather_idx, segment_ids):
    mesh = plsc.VectorSubcoreMesh(core_axis_name="c", subcore_axis_name="s")
    f = pl.kernel(
        _segmax_body,
        out_type=jax.ShapeDtypeStruct((_N_SEG, _D), jnp.float32),
        mesh=mesh,
        scratch_types=[
            pltpu.VMEM((_CHUNK,), jnp.int32),
            pltpu.VMEM((_CHUNK,), jnp.int32),
            pltpu.VMEM((_CHUNK,), jnp.int32),
            pltpu.VMEM((_CHUNK,), jnp.int32),
            pltpu.VMEM((_CHUNK, _D), jnp.float32),
            pltpu.VMEM((_CHUNK, _D), jnp.float32),
            pltpu.VMEM((_CHUNK,), jnp.int32),
            pltpu.VMEM((_SEG_PER_W, _D), jnp.float32),
            pltpu.VMEM((_L,), jnp.int32),
            pltpu.VMEM((_L,), jnp.int32),
            pltpu.SemaphoreType.DMA,
            pltpu.SemaphoreType.DMA,
            pltpu.SemaphoreType.DMA,
            pltpu.SemaphoreType.DMA,
            pltpu.SemaphoreType.DMA,
        ],
        compiler_params=pltpu.CompilerParams(needs_layout_passes=False),
    )
    return f(x, gather_idx, segment_ids)


def kernel(x, gather_idx, segment_ids):
    seg_max = _segmax(x, gather_idx, segment_ids)
    half = _N_SEG // 2
    return jnp.concatenate((seg_max[:half], seg_max[half:]), axis=1)


# speedup vs baseline: 1.0949x; 1.0949x over previous
import jax
import jax.numpy as jnp
from jax import lax
from jax.experimental import pallas as pl
from jax.experimental.pallas import tpu as pltpu
from jax.experimental.pallas import tpu_sc as plsc

_N_NODES = 10000
_N_GATHER = 320000
_N_SEG = 8192
_D = 128

_NC = 2
_NS = 16
_NW = _NC * _NS
_SEG_PER_W = _N_SEG // _NW
_CHUNK = 128
_L = 16
_G = _CHUNK // _L

_INT_MAX = 2**31 - 1


def _iota16():
    return lax.iota(jnp.int32, _L)


def _lower_bound(seg_hbm, probe_ref, pidx_ref, sem, t_splat):
    lo = 0
    for s16 in (65536, 4096, 256, 16, 1):
        probe = lo + (_iota16() + 1) * s16 - 1
        pidx_ref[...] = jnp.minimum(probe, _N_GATHER - 1)
        pltpu.async_copy(seg_hbm.at[pidx_ref], probe_ref, sem).wait()
        vals = jnp.where(probe < _N_GATHER, probe_ref[...],
                         jnp.full((_L,), _INT_MAX, jnp.int32))
        c = jnp.sum((vals < t_splat).astype(jnp.int32))
        lo = lo + c * s16
    return lo


def _segmax_body(x_hbm, gid_hbm, seg_hbm, out_hbm,
                 gid0, gid1, seg0, seg1, rows0, rows1, segst, acc,
                 probe, pidx, ssem, gsem0, gsem1, isem0, isem1):
    cid = lax.axis_index("c")
    sid = lax.axis_index("s")
    wid = sid * _NC + cid
    base = wid * _SEG_PER_W
    base_v = jnp.full((_L,), base, jnp.int32)

    gid_b = (gid0, gid1)
    seg_b = (seg0, seg1)
    rows_b = (rows0, rows1)
    gsem = (gsem0, gsem1)
    isem = (isem0, isem1)

    neg = jnp.full((_L,), -jnp.inf, jnp.float32)

    def init_body(i, _):
        for j in range(_D // _L):
            acc[i, pl.ds(j * _L, _L)] = neg
        return 0

    lax.fori_loop(0, _SEG_PER_W, init_body, 0)

    lo = _lower_bound(seg_hbm, probe, pidx, ssem, base_v)
    hi = _lower_bound(seg_hbm, probe, pidx, ssem, base_v + _SEG_PER_W)
    lo_al = pl.multiple_of((lo // 8) * 8, 8)
    nch = (hi - lo_al + (_CHUNK - 1)) // _CHUNK

    def r0_of(c):
        return pl.multiple_of(
            jnp.minimum(lo_al + c * _CHUNK, _N_GATHER - _CHUNK), 8)

    def issue_idx(c, b):
        r0 = r0_of(c)
        pltpu.async_copy(gid_hbm.at[pl.ds(r0, _CHUNK)], gid_b[b], isem[b])
        pltpu.async_copy(seg_hbm.at[pl.ds(r0, _CHUNK)], seg_b[b], isem[b])

    def wait_idx(c, b):
        r0 = r0_of(c)
        pltpu.make_async_copy(gid_hbm.at[pl.ds(r0, _CHUNK)], gid_b[b],
                              isem[b]).wait()
        pltpu.make_async_copy(seg_hbm.at[pl.ds(r0, _CHUNK)], seg_b[b],
                              isem[b]).wait()

    def issue_gather(b):
        pltpu.async_copy(x_hbm.at[gid_b[b]], rows_b[b], gsem[b])

    def wait_gather(b):
        pltpu.make_async_copy(x_hbm.at[gid_b[b]], rows_b[b], gsem[b]).wait()

    @pl.when(nch > 0)
    def _prime0():
        issue_idx(0, 0)

    @pl.when(nch > 1)
    def _prime1():
        issue_idx(1, 1)

    @pl.when(nch > 0)
    def _prime2():
        wait_idx(0, 0)
        issue_gather(0)

    def compute(rows):
        def group_body(g, _):
            gb = g * _L
            seg_vec = segst[pl.ds(gb, _L)]
            s0 = seg_vec[0]
            s15 = seg_vec[_L - 1]
            off0 = s0 - base
            ok0 = (off0 >= 0) & (off0 < _SEG_PER_W)

            @pl.when((s0 == s15) & ok0)
            def _fast():
                for j in range(_D // _L):
                    js = pl.ds(j * _L, _L)
                    vs = [rows[gb + r, js] for r in range(_L)]
                    while len(vs) > 1:
                        vs = [jnp.maximum(vs[i], vs[i + 1])
                              for i in range(0, len(vs), 2)]
                    acc[off0, js] = jnp.maximum(acc[off0, js], vs[0])

            @pl.when(s0 != s15)
            def _slow():
                for r in range(_L):
                    off = seg_vec[r] - base

                    @pl.when((off >= 0) & (off < _SEG_PER_W))
                    def _row(r=r, off=off):
                        for j in range(_D // _L):
                            js = pl.ds(j * _L, _L)
                            acc[off, js] = jnp.maximum(acc[off, js],
                                                       rows[gb + r, js])

            return 0

        lax.fori_loop(0, _G, group_body, 0)

    def pair_body(cc, _):
        for b in (0, 1):
            c = cc * 2 + b

            @pl.when(c < nch)
            def _iter(c=c, b=b):
                nb = 1 - b
                wait_gather(b)

                @pl.when(c + 1 < nch)
                def _wi():
                    wait_idx(c + 1, nb)

                for g in range(_G):
                    segst[pl.ds(g * _L, _L)] = seg_b[b][pl.ds(g * _L, _L)]

                @pl.when(c + 1 < nch)
                def _ig():
                    issue_gather(nb)

                @pl.when(c + 2 < nch)
                def _ii():
                    issue_idx(c + 2, b)

                compute(rows_b[b])

        return 0

    lax.fori_loop(0, (nch + 1) // 2, pair_body, 0)

    pltpu.sync_copy(acc, out_hbm.at[pl.ds(base, _SEG_PER_W)])


@jax.jit
def _segmax(x, gather_idx, segment_ids):
    mesh = plsc.VectorSubcoreMesh(core_axis_name="c", subcore_axis_name="s")
    f = pl.kernel(
        _segmax_body,
        out_type=jax.ShapeDtypeStruct((_N_SEG, _D), jnp.float32),
        mesh=mesh,
        scratch_types=[
            pltpu.VMEM((_CHUNK,), jnp.int32),
            pltpu.VMEM((_CHUNK,), jnp.int32),
            pltpu.VMEM((_CHUNK,), jnp.int32),
            pltpu.VMEM((_CHUNK,), jnp.int32),
            pltpu.VMEM((_CHUNK, _D), jnp.float32),
            pltpu.VMEM((_CHUNK, _D), jnp.float32),
            pltpu.VMEM((_CHUNK,), jnp.int32),
            pltpu.VMEM((_SEG_PER_W, _D), jnp.float32),
            pltpu.VMEM((_L,), jnp.int32),
            pltpu.VMEM((_L,), jnp.int32),
            pltpu.SemaphoreType.DMA,
            pltpu.SemaphoreType.DMA,
            pltpu.SemaphoreType.DMA,
            pltpu.SemaphoreType.DMA,
            pltpu.SemaphoreType.DMA,
        ],
        compiler_params=pltpu.CompilerParams(needs_layout_passes=False),
    )
    return f(x, gather_idx, segment_ids)


def kernel(x, gather_idx, segment_ids):
    seg_max = _segmax(x, gather_idx, segment_ids)
    half = _N_SEG // 2
    return jnp.concatenate((seg_max[:half], seg_max[half:]), axis=1)
